# Initial kernel scaffold; baseline (speedup 1.0000x reference)
import jax
import jax.numpy as jnp
from jax.experimental import pallas as pl


def kernel(x, adj1_indices, adj1_values, adj2_indices, adj2_values, W1, W_out, b_out):
    raise NotImplementedError("write your pallas kernel here")


# speedup vs baseline: 4.0153x; 4.0153x over previous
import functools

import jax
import jax.numpy as jnp
from jax import lax
from jax.experimental import pallas as pl
from jax.experimental.pallas import tpu as pltpu
from jax.experimental.pallas import tpu_sc as plsc

_N = 10000
_D = 128
_K = 128
_NSUB = 16
_PAD = _NSUB * _K
_RPT = 624


def _mm_in_body(x_ref, w_ref, o_ref):
    o_ref[...] = jnp.dot(x_ref[...], w_ref[...],
                         preferred_element_type=jnp.float32)


def _dense_in(x, wt):
    return pl.pallas_call(
        _mm_in_body,
        grid=(10,),
        in_specs=[pl.BlockSpec((_N // 10, _D), lambda i: (i, 0)),
                  pl.BlockSpec((_D, _D), lambda i: (0, 0))],
        out_specs=pl.BlockSpec((_N // 10, _D), lambda i: (i, 0)),
        out_shape=jax.ShapeDtypeStruct((_N, _D), jnp.float32),
    )(x, wt)


def _proj_body(h0_ref, h1_ref, h2_ref, h3_ref, h4_ref, wt_ref, b_ref, o_ref):
    acc = jnp.dot(h0_ref[...], wt_ref[0:128, :],
                  preferred_element_type=jnp.float32)
    acc = acc + jnp.dot(h1_ref[...], wt_ref[128:256, :],
                        preferred_element_type=jnp.float32)
    acc = acc + jnp.dot(h2_ref[...], wt_ref[256:384, :],
                        preferred_element_type=jnp.float32)
    acc = acc + jnp.dot(h3_ref[...], wt_ref[384:512, :],
                        preferred_element_type=jnp.float32)
    acc = acc + jnp.dot(h4_ref[...], wt_ref[512:640, :],
                        preferred_element_type=jnp.float32)
    o_ref[...] = acc + b_ref[...]


def _proj(h0, h1, h2, h3, h4, wt, b2):
    blk = _N // 10
    hspec = pl.BlockSpec((blk, _D), lambda i: (i, 0))
    return pl.pallas_call(
        _proj_body,
        grid=(10,),
        in_specs=[hspec, hspec, hspec, hspec, hspec,
                  pl.BlockSpec((640, 64), lambda i: (0, 0)),
                  pl.BlockSpec((1, 64), lambda i: (0, 0))],
        out_specs=pl.BlockSpec((blk, 64), lambda i: (i, 0)),
        out_shape=jax.ShapeDtypeStruct((_N, 64), jnp.float32),
    )(h0, h1, h2, h3, h4, wt, b2)


def _sc_pair_body(h, ra, ca, va, rb, cb, vb, out_a, out_b,
                  colv, rowv, valv, rowsv, acc, sem):
    cid = lax.axis_index("c")
    sid = lax.axis_index("s")

    def _zrow(r, carry):
        for j in range(8):
            rowsv[r, pl.ds(16 * j, 16)] = jnp.zeros((16,), jnp.float32)
        return carry
    lax.fori_loop(0, _K, _zrow, 0)
    for t in range(6):
        pltpu.sync_copy(rowsv.at[pl.ds(0, 104)],
                        acc.at[pl.ds(sid * _RPT + t * 104, 104)])

    @pl.when(sid == _NSUB - 1)
    def _():
        pltpu.sync_copy(rowsv.at[pl.ds(0, 16)],
                        acc.at[pl.ds(_NSUB * _RPT, 16)])
    plsc.subcore_barrier()

    def _run(rows_h, cols_h, vals_h, out_h):
        ept = vals_h.shape[0] // _NSUB
        nch = ept // _K
        tbase = sid * ept

        def _chunk(ci, carry):
            base = tbase + ci * _K
            pltpu.sync_copy(cols_h.at[pl.ds(base, _K)], colv)
            pltpu.sync_copy(rows_h.at[pl.ds(base, _K)], rowv)
            pltpu.sync_copy(vals_h.at[pl.ds(base, _K)], valv)
            pltpu.async_copy(h.at[colv], rowsv, sem).wait()

            def _group(g, c2):
                v16 = valv[pl.ds(16 * g, 16)]

                def _edge(el, c3):
                    e = 16 * g + el
                    lane = (jnp.zeros((16,), jnp.int32) + el)[:, None]
                    vsp = lax.gather(
                        v16, lane,
                        lax.GatherDimensionNumbers(
                            offset_dims=(), collapsed_slice_dims=(0,),
                            start_index_map=(0,)),
                        slice_sizes=(1,),
                        mode=lax.GatherScatterMode.PROMISE_IN_BOUNDS)
                    for j in range(8):
                        sl = rowsv[e, pl.ds(16 * j, 16)]
                        rowsv[e, pl.ds(16 * j, 16)] = sl * vsp
                    return c3
                lax.fori_loop(0, 16, _edge, 0)
                return c2
            lax.fori_loop(0, 8, _group, 0)
            pltpu.sync_copy(rowsv, acc.at[rowv], add=True)
            return carry
        lax.fori_loop(0, nch, _chunk, 0)

        plsc.subcore_barrier()
        pltpu.sync_copy(acc.at[pl.ds(sid * _RPT, _RPT)],
                        out_h.at[pl.ds(sid * _RPT, _RPT)])

        @pl.when(sid == _NSUB - 1)
        def _():
            pltpu.sync_copy(acc.at[pl.ds(_NSUB * _RPT, 16)],
                            out_h.at[pl.ds(_NSUB * _RPT, 16)])

    @pl.when(cid == 0)
    def _():
        _run(ra, ca, va, out_a)

    @pl.when(cid == 1)
    def _():
        _run(rb, cb, vb, out_b)


_spmm_pair = functools.partial(
    pl.kernel,
    mesh=plsc.VectorSubcoreMesh(core_axis_name="c", subcore_axis_name="s"),
    out_type=(jax.ShapeDtypeStruct((_N, _D), jnp.float32),
              jax.ShapeDtypeStruct((_N, _D), jnp.float32)),
    scratch_types=[
        pltpu.VMEM((_K,), jnp.int32),
        pltpu.VMEM((_K,), jnp.int32),
        pltpu.VMEM((_K,), jnp.float32),
        pltpu.VMEM((_K, _D), jnp.float32),
        pltpu.VMEM_SHARED((_N, _D), jnp.float32),
        pltpu.SemaphoreType.DMA,
    ],
)(_sc_pair_body)


def _pad_edges(idx, val):
    e = val.shape[0]
    ep = -(-e // _PAD) * _PAD
    pad = ep - e
    return (jnp.pad(idx[0], (0, pad)), jnp.pad(idx[1], (0, pad)),
            jnp.pad(val, (0, pad)))


def kernel(x, adj1_indices, adj1_values, adj2_indices, adj2_values,
           W1, W_out, b_out):
    r1, c1, v1 = _pad_edges(adj1_indices, adj1_values)
    r2, c2, v2 = _pad_edges(adj2_indices, adj2_values)
    h0 = _dense_in(x, W1.T)
    h1, h2 = _spmm_pair(h0, r1, c1, v1, r2, c2, v2)
    h3, h4 = _spmm_pair(h1, r1, c1, v1, r2, c2, v2)
    return _proj(h0, h1, h2, h3, h4, W_out.T, b_out.reshape(1, 64))


# speedup vs baseline: 4.1492x; 1.0333x over previous
import functools

import jax
import jax.numpy as jnp
from jax import lax
from jax.experimental import pallas as pl
from jax.experimental.pallas import tpu as pltpu
from jax.experimental.pallas import tpu_sc as plsc

_N = 10000
_D = 128
_K = 96
_NSUB = 16
_BLK = 16
_PAD = _NSUB * 8 * _K
_RPT = 624


def _mm_in_body(x_ref, w_ref, o_ref):
    o_ref[...] = jnp.dot(x_ref[...], w_ref[...],
                         preferred_element_type=jnp.float32)


def _dense_in(x, wt):
    return pl.pallas_call(
        _mm_in_body,
        grid=(10,),
        in_specs=[pl.BlockSpec((_N // 10, _D), lambda i: (i, 0)),
                  pl.BlockSpec((_D, _D), lambda i: (0, 0))],
        out_specs=pl.BlockSpec((_N // 10, _D), lambda i: (i, 0)),
        out_shape=jax.ShapeDtypeStruct((_N, _D), jnp.float32),
    )(x, wt)


def _proj_body(h0_ref, h1_ref, h2_ref, h3_ref, h4_ref, wt_ref, b_ref, o_ref):
    acc = jnp.dot(h0_ref[...], wt_ref[0:128, :],
                  preferred_element_type=jnp.float32)
    acc = acc + jnp.dot(h1_ref[...], wt_ref[128:256, :],
                        preferred_element_type=jnp.float32)
    acc = acc + jnp.dot(h2_ref[...], wt_ref[256:384, :],
                        preferred_element_type=jnp.float32)
    acc = acc + jnp.dot(h3_ref[...], wt_ref[384:512, :],
                        preferred_element_type=jnp.float32)
    acc = acc + jnp.dot(h4_ref[...], wt_ref[512:640, :],
                        preferred_element_type=jnp.float32)
    o_ref[...] = acc + b_ref[...]


def _proj(h0, h1, h2, h3, h4, wt, b2):
    blk = _N // 10
    hspec = pl.BlockSpec((blk, _D), lambda i: (i, 0))
    return pl.pallas_call(
        _proj_body,
        grid=(10,),
        in_specs=[hspec, hspec, hspec, hspec, hspec,
                  pl.BlockSpec((640, 64), lambda i: (0, 0)),
                  pl.BlockSpec((1, 64), lambda i: (0, 0))],
        out_specs=pl.BlockSpec((blk, 64), lambda i: (i, 0)),
        out_shape=jax.ShapeDtypeStruct((_N, 64), jnp.float32),
    )(h0, h1, h2, h3, h4, wt, b2)


def _lane_splat(v16, el):
    lane = (jnp.zeros((16,), jnp.int32) + el)[:, None]
    return lax.gather(
        v16, lane,
        lax.GatherDimensionNumbers(offset_dims=(), collapsed_slice_dims=(0,),
                                   start_index_map=(0,)),
        slice_sizes=(1,),
        mode=lax.GatherScatterMode.PROMISE_IN_BOUNDS)


def _sc_pair_body(h, ra, ca, va, rb, cb, vb, out_a, out_b,
                  colb, rowb, valb, r0, r1, r2, acc,
                  sg0, sg1, sg2, ss0, ss1, ss2):
    cid = lax.axis_index("c")
    sid = lax.axis_index("s")
    bufs = (r0, r1, r2)
    gsems = (sg0, sg1, sg2)
    ssems = (ss0, ss1, ss2)

    def _zrow(r, carry):
        for j in range(8):
            r0[r, pl.ds(16 * j, 16)] = jnp.zeros((16,), jnp.float32)
        return carry
    lax.fori_loop(0, _K, _zrow, 0)
    for t in range(6):
        pltpu.sync_copy(r0.at[pl.ds(0, 96)],
                        acc.at[pl.ds(sid * _RPT + t * 96, 96)])
    pltpu.sync_copy(r0.at[pl.ds(0, 48)],
                    acc.at[pl.ds(sid * _RPT + 576, 48)])

    @pl.when(sid == _NSUB - 1)
    def _():
        pltpu.sync_copy(r0.at[pl.ds(0, 16)],
                        acc.at[pl.ds(_NSUB * _RPT, 16)])

    def _run(rows2d, cols2d, vals2d, out_h):
        nch = cols2d.shape[0] // _NSUB
        cbase = sid * nch

        def _parity(c):
            return (c // _BLK) % 2

        def _slot(c):
            return c % _BLK

        def _load_block(c0):
            p = _parity(c0)
            b8 = pl.multiple_of(cbase + c0, 8)
            pltpu.sync_copy(cols2d.at[pl.ds(b8, _BLK)], colb.at[p])
            pltpu.sync_copy(rows2d.at[pl.ds(b8, _BLK)], rowb.at[p])
            pltpu.sync_copy(vals2d.at[pl.ds(b8, _BLK)], valb.at[p])

        def _start_gather(c, k):
            pltpu.async_copy(h.at[colb.at[_parity(c), _slot(c)]],
                             bufs[k], gsems[k])

        def _wait_gather(c, k):
            pltpu.make_async_copy(h.at[colb.at[_parity(c), _slot(c)]],
                                  bufs[k], gsems[k]).wait()

        def _start_scatter(c, k):
            pltpu.async_copy(bufs[k],
                             acc.at[rowb.at[_parity(c), _slot(c)]],
                             ssems[k], add=True)

        def _wait_scatter(c, k):
            pltpu.make_async_copy(bufs[k],
                                  acc.at[rowb.at[_parity(c), _slot(c)]],
                                  ssems[k]).wait()

        def _step(ci, k):
            kp1 = (k + 1) % 3

            @pl.when(ci >= 2)
            def _():
                _wait_scatter(ci - 2, kp1)

            @pl.when(jnp.logical_and((ci + 1) % _BLK == 0, ci + 1 < nch))
            def _():
                _load_block(ci + 1)

            @pl.when(ci + 1 < nch)
            def _():
                _start_gather(ci + 1, kp1)

            _wait_gather(ci, k)

            p = _parity(ci)
            j = _slot(ci)

            def _group(g, carry):
                v16 = valb[p, j, pl.ds(16 * g, 16)]
                for el in range(16):
                    e = 16 * g + el
                    vsp = _lane_splat(v16, el)
                    for q in range(8):
                        sl = bufs[k][e, pl.ds(16 * q, 16)]
                        bufs[k][e, pl.ds(16 * q, 16)] = sl * vsp
                return carry
            lax.fori_loop(0, _K // 16, _group, 0)

            _start_scatter(ci, k)

        _load_block(0)
        _start_gather(0, 0)
        plsc.subcore_barrier()

        ntr, rem = divmod(nch, 3)

        def _triple(ti, carry):
            _step(3 * ti, 0)
            _step(3 * ti + 1, 1)
            _step(3 * ti + 2, 2)
            return carry
        lax.fori_loop(0, ntr, _triple, 0)
        for q in range(rem):
            _step(jnp.int32(3 * ntr + q), q)

        _wait_scatter(jnp.int32(nch - 2), (nch - 2) % 3)
        _wait_scatter(jnp.int32(nch - 1), (nch - 1) % 3)
        plsc.subcore_barrier()

        pltpu.sync_copy(acc.at[pl.ds(sid * _RPT, _RPT)],
                        out_h.at[pl.ds(sid * _RPT, _RPT)])

        @pl.when(sid == _NSUB - 1)
        def _():
            pltpu.sync_copy(acc.at[pl.ds(_NSUB * _RPT, 16)],
                            out_h.at[pl.ds(_NSUB * _RPT, 16)])

    @pl.when(cid == 0)
    def _():
        _run(ra, ca, va, out_a)

    @pl.when(cid == 1)
    def _():
        _run(rb, cb, vb, out_b)


_spmm_pair = functools.partial(
    pl.kernel,
    mesh=plsc.VectorSubcoreMesh(core_axis_name="c", subcore_axis_name="s"),
    out_type=(jax.ShapeDtypeStruct((_N, _D), jnp.float32),
              jax.ShapeDtypeStruct((_N, _D), jnp.float32)),
    scratch_types=[
        pltpu.VMEM((2, _BLK, _K), jnp.int32),
        pltpu.VMEM((2, _BLK, _K), jnp.int32),
        pltpu.VMEM((2, _BLK, _K), jnp.float32),
        pltpu.VMEM((_K, _D), jnp.float32),
        pltpu.VMEM((_K, _D), jnp.float32),
        pltpu.VMEM((_K, _D), jnp.float32),
        pltpu.VMEM_SHARED((_N, _D), jnp.float32),
        pltpu.SemaphoreType.DMA,
        pltpu.SemaphoreType.DMA,
        pltpu.SemaphoreType.DMA,
        pltpu.SemaphoreType.DMA,
        pltpu.SemaphoreType.DMA,
        pltpu.SemaphoreType.DMA,
    ],
)(_sc_pair_body)


def _prep_edges(idx, val):
    e = val.shape[0]
    ep = -(-e // _PAD) * _PAD
    pad = ep - e
    rows = jnp.pad(idx[0], (0, pad)).reshape(-1, _K)
    cols = jnp.pad(idx[1], (0, pad)).reshape(-1, _K)
    vals = jnp.pad(val, (0, pad)).reshape(-1, _K)
    return rows, cols, vals


def kernel(x, adj1_indices, adj1_values, adj2_indices, adj2_values,
           W1, W_out, b_out):
    r1, c1, v1 = _prep_edges(adj1_indices, adj1_values)
    r2, c2, v2 = _prep_edges(adj2_indices, adj2_values)
    h0 = _dense_in(x, W1.T)
    h1, h2 = _spmm_pair(h0, r1, c1, v1, r2, c2, v2)
    h3, h4 = _spmm_pair(h1, r1, c1, v1, r2, c2, v2)
    return _proj(h0, h1, h2, h3, h4, W_out.T, b_out.reshape(1, 64))
